# baseline (speedup 1.0000x reference)
import functools
import math

import jax
import jax.numpy as jnp
from jax import lax
from jax.experimental import pallas as pl
from jax.experimental.pallas import tpu as pltpu
from jax.experimental.pallas import tpu_sc as plsc

HID = 64
PADW = 128
SCALE = math.sqrt(HID)

NC = 2
NS = 16
NW = NC * NS
BBLK = 128
TBLK = 2048
NBUF = 6


def _tpose_body(x_ref, o_ref):
    for j in range(TBLK // 128):
        sl = pl.ds(j * 128, 128)
        o_ref[sl, 0:HID] = jnp.transpose(x_ref[:, sl]) * SCALE
    o_ref[:, HID:PADW] = jnp.zeros((TBLK, PADW - HID), jnp.float32)


def _make_tpose(vocab):
    return pl.pallas_call(
        _tpose_body,
        grid=(vocab // TBLK,),
        in_specs=[pl.BlockSpec((HID, TBLK), lambda g: (0, g))],
        out_specs=pl.BlockSpec((TBLK, PADW), lambda g: (g, 0)),
        out_shape=jax.ShapeDtypeStruct((vocab, PADW), jnp.float32),
    )


def _emb_body(seq, idsT_hbm, table_hbm, out_hbm,
              idx_v, g0, g1, g2, g3, g4, g5, isem,
              gs0, gs1, gs2, gs3, gs4, gs5,
              ws0, ws1, ws2, ws3, ws4, ws5):
    wid = lax.axis_index("s") * NC + lax.axis_index("c")
    b0 = wid * BBLK
    gbuf = (g0, g1, g2, g3, g4, g5)
    gsem = (gs0, gs1, gs2, gs3, gs4, gs5)
    wsem = (ws0, ws1, ws2, ws3, ws4, ws5)

    for t in range(seq // 8):
        pltpu.async_copy(idsT_hbm.at[pl.ds(t * 8, 8), pl.ds(b0, BBLK)],
                         idx_v.at[pl.ds(t * 8, 8)], isem)
    for t in range(seq // 8):
        pltpu.make_async_copy(idsT_hbm.at[pl.ds(0, 8), pl.ds(0, BBLK)],
                              idx_v.at[pl.ds(0, 8)], isem).wait()

    def start_gather(s, b):
        pltpu.async_copy(table_hbm.at[idx_v.at[s]], gbuf[b], gsem[b])

    def wait_gather(b):
        pltpu.make_async_copy(table_hbm.at[idx_v.at[0]], gbuf[b], gsem[b]).wait()

    def start_write(s, b):
        pltpu.async_copy(gbuf[b], out_hbm.at[s, pl.ds(b0, BBLK)], wsem[b])

    def wait_write(b):
        pltpu.make_async_copy(gbuf[b], out_hbm.at[0, pl.ds(0, BBLK)],
                              wsem[b]).wait()

    for b in range(NBUF):
        start_gather(b, b)

    for s in (0, 1, 2):
        wait_gather(s)
        start_write(s, s)

    @pl.loop(3, seq - 5, step=NBUF)
    def _(s0_):
        for k in range(NBUF):
            s = s0_ + k
            b = (3 + k) % NBUF
            bprev = k % NBUF
            wait_gather(b)
            start_write(s, b)
            wait_write(bprev)
            start_gather(s + 3, bprev)

    for s in range(seq - 5, seq):
        b = s % NBUF
        wait_gather(b)
        start_write(s, b)
        wait_write((s - 3) % NBUF)
        if s + 3 < seq:
            start_gather(s + 3, (s + 3) % NBUF)
    for s in range(seq - 3, seq):
        wait_write(s % NBUF)


def _make_emb(seq, n_batch):
    assert n_batch == NW * BBLK
    mesh = plsc.VectorSubcoreMesh(core_axis_name="c", subcore_axis_name="s")
    return pl.kernel(
        functools.partial(_emb_body, seq),
        out_type=jax.ShapeDtypeStruct((seq, n_batch, PADW), jnp.float32),
        mesh=mesh,
        scratch_types=[
            pltpu.VMEM((seq, BBLK), jnp.int32),
            pltpu.VMEM((BBLK, PADW), jnp.float32),
            pltpu.VMEM((BBLK, PADW), jnp.float32),
            pltpu.VMEM((BBLK, PADW), jnp.float32),
            pltpu.VMEM((BBLK, PADW), jnp.float32),
            pltpu.VMEM((BBLK, PADW), jnp.float32),
            pltpu.VMEM((BBLK, PADW), jnp.float32),
            pltpu.SemaphoreType.DMA,
            pltpu.SemaphoreType.DMA,
            pltpu.SemaphoreType.DMA,
            pltpu.SemaphoreType.DMA,
            pltpu.SemaphoreType.DMA,
            pltpu.SemaphoreType.DMA,
            pltpu.SemaphoreType.DMA,
            pltpu.SemaphoreType.DMA,
            pltpu.SemaphoreType.DMA,
            pltpu.SemaphoreType.DMA,
            pltpu.SemaphoreType.DMA,
            pltpu.SemaphoreType.DMA,
            pltpu.SemaphoreType.DMA,
        ],
        compiler_params=pltpu.CompilerParams(use_tc_tiling_on_sc=True,
                                             needs_layout_passes=False),
    )


def kernel(input_ids, table):
    n_batch, seq = input_ids.shape
    idsT = input_ids.T.astype(jnp.int32)
    tscaled = jnp.pad(table, ((0, 0), (0, PADW - HID))) * SCALE
    out_wide = _make_emb(seq, n_batch)(idsT, tscaled)
    return out_wide.transpose(1, 0, 2)[:, :, :HID]


# speedup vs baseline: 1.0007x; 1.0007x over previous
import functools
import math

import jax
import jax.numpy as jnp
from jax import lax
from jax.experimental import pallas as pl
from jax.experimental.pallas import tpu as pltpu
from jax.experimental.pallas import tpu_sc as plsc

HID = 64
PADW = 128
SCALE = math.sqrt(HID)

NC = 2
NS = 16
NW = NC * NS
BBLK = 128
NBUF = 6


def _emb_body(seq, idsT_hbm, table_hbm, out_hbm,
              idx_v, g0, g1, g2, g3, g4, g5, isem,
              gs0, gs1, gs2, gs3, gs4, gs5,
              ws0, ws1, ws2, ws3, ws4, ws5):
    wid = lax.axis_index("s") * NC + lax.axis_index("c")
    b0 = wid * BBLK
    gbuf = (g0, g1, g2, g3, g4, g5)
    gsem = (gs0, gs1, gs2, gs3, gs4, gs5)
    wsem = (ws0, ws1, ws2, ws3, ws4, ws5)

    for t in range(seq // 8):
        pltpu.async_copy(idsT_hbm.at[pl.ds(t * 8, 8), pl.ds(b0, BBLK)],
                         idx_v.at[pl.ds(t * 8, 8)], isem)
    for t in range(seq // 8):
        pltpu.make_async_copy(idsT_hbm.at[pl.ds(0, 8), pl.ds(0, BBLK)],
                              idx_v.at[pl.ds(0, 8)], isem).wait()

    def start_gather(s, b):
        pltpu.async_copy(table_hbm.at[idx_v.at[s]], gbuf[b], gsem[b])

    def wait_gather(b):
        pltpu.make_async_copy(table_hbm.at[idx_v.at[0]], gbuf[b], gsem[b]).wait()

    def start_write(s, b):
        pltpu.async_copy(gbuf[b], out_hbm.at[s, pl.ds(b0, BBLK)], wsem[b])

    def wait_write(b):
        pltpu.make_async_copy(gbuf[b], out_hbm.at[0, pl.ds(0, BBLK)],
                              wsem[b]).wait()

    for b in range(NBUF):
        start_gather(b, b)

    for s in (0, 1, 2):
        wait_gather(s)
        start_write(s, s)

    @pl.loop(3, seq - 5, step=NBUF)
    def _(s0_):
        for k in range(NBUF):
            s = s0_ + k
            b = (3 + k) % NBUF
            bprev = k % NBUF
            wait_gather(b)
            start_write(s, b)
            wait_write(bprev)
            start_gather(s + 3, bprev)

    for s in range(seq - 5, seq):
        b = s % NBUF
        wait_gather(b)
        start_write(s, b)
        wait_write((s - 3) % NBUF)
        if s + 3 < seq:
            start_gather(s + 3, (s + 3) % NBUF)
    for s in range(seq - 3, seq):
        wait_write(s % NBUF)


def _make_emb(seq, n_batch):
    assert n_batch == NW * BBLK
    mesh = plsc.VectorSubcoreMesh(core_axis_name="c", subcore_axis_name="s")
    return pl.kernel(
        functools.partial(_emb_body, seq),
        out_type=jax.ShapeDtypeStruct((seq, n_batch, PADW), jnp.float32),
        mesh=mesh,
        scratch_types=[
            pltpu.VMEM((seq, BBLK), jnp.int32),
            pltpu.VMEM((BBLK, PADW), jnp.float32),
            pltpu.VMEM((BBLK, PADW), jnp.float32),
            pltpu.VMEM((BBLK, PADW), jnp.float32),
            pltpu.VMEM((BBLK, PADW), jnp.float32),
            pltpu.VMEM((BBLK, PADW), jnp.float32),
            pltpu.VMEM((BBLK, PADW), jnp.float32),
            pltpu.SemaphoreType.DMA,
            pltpu.SemaphoreType.DMA,
            pltpu.SemaphoreType.DMA,
            pltpu.SemaphoreType.DMA,
            pltpu.SemaphoreType.DMA,
            pltpu.SemaphoreType.DMA,
            pltpu.SemaphoreType.DMA,
            pltpu.SemaphoreType.DMA,
            pltpu.SemaphoreType.DMA,
            pltpu.SemaphoreType.DMA,
            pltpu.SemaphoreType.DMA,
            pltpu.SemaphoreType.DMA,
            pltpu.SemaphoreType.DMA,
        ],
        compiler_params=pltpu.CompilerParams(use_tc_tiling_on_sc=True,
                                             needs_layout_passes=False),
    )


def kernel(input_ids, table):
    n_batch, seq = input_ids.shape
    idsT = input_ids.T.astype(jnp.int32)
    tscaled = jnp.pad(table, ((0, 0), (0, PADW - HID))) * SCALE
    out_wide = _make_emb(seq, n_batch)(idsT, tscaled)
    return out_wide.transpose(1, 0, 2)[:, :, :HID]
